# baseline (speedup 1.0000x reference)
import functools

import jax
import jax.numpy as jnp
from jax import lax
from jax.experimental import pallas as pl
from jax.experimental.pallas import tpu as pltpu
from jax.experimental.pallas import tpu_sc as plsc

_T = 800000
_N = 10000
_H = 256
_HW = _H // 2
_R = 16
_REG = 0.01
_C = 64
_NW = 32
_L = 16
_NCH_W = 392
_NB = _NCH_W // 4
_NCH = _NCH_W * _NW
_TP = _NCH * _C
_WREP = 256


def _sc_body(hidx_hbm, ctab_hbm, out_hbm,
             h_a, h_b, rows_a, rows_b, rows_c2, rows_d2,
             scb_a, scb_b, sbuf,
             sem_a, sem_b, sem_c2, sem_d2, sem_ia, sem_ib, osem_a, osem_b):
    wid = lax.axis_index("s") * 2 + lax.axis_index("c")
    lane = lax.broadcasted_iota(jnp.int32, (_L,), 0)

    h_bufs = (h_a, h_b)
    row_bufs = (rows_a, rows_b, rows_c2, rows_d2)
    row_sems = (sem_a, sem_b, sem_c2, sem_d2)
    scb_bufs = (scb_a, scb_b)
    out_sems = (osem_a, osem_b)

    pltpu.sync_copy(hidx_hbm.at[wid, pl.ds(0, 4)], h_a)
    pltpu.sync_copy(hidx_hbm.at[wid, pl.ds(4, 4)], h_b)
    pltpu.async_copy(ctab_hbm.at[h_a.at[0]], rows_a, sem_a)
    pltpu.async_copy(ctab_hbm.at[h_a.at[1]], rows_b, sem_b)

    def wait_idx(par):
        sem = sem_ia if par == 0 else sem_ib
        pltpu.make_async_copy(hidx_hbm.at[wid, pl.ds(0, 4)],
                              h_bufs[par], sem).wait()

    def issue_idx(par, batch_base):
        sem = sem_ia if par == 0 else sem_ib
        pltpu.async_copy(hidx_hbm.at[wid, pl.ds(batch_base, 4)],
                         h_bufs[par], sem)

    def issue_streams(src_par, slot, dst_p):
        pltpu.async_copy(ctab_hbm.at[h_bufs[src_par].at[slot]],
                         row_bufs[dst_p], row_sems[dst_p])

    def wait_rows(p):
        pltpu.make_async_copy(ctab_hbm.at[pl.ds(0, 2 * _C)],
                              row_bufs[p], row_sems[p]).wait()

    def compute_chunk(p, scb, slot):
        rows_c = row_bufs[p]

        def group_body(g, carry):
            score_vec = jnp.zeros((_L,), jnp.float32)
            himask = jnp.full((_L,), -65536, jnp.int32)
            for i2 in range(_L):
                t = g * _L + i2
                acc = jnp.zeros((_L,), jnp.float32)
                for j in range(_HW // _L):
                    s2 = rows_c[t, pl.ds(j * _L, _L)]
                    o2 = rows_c[_C + t, pl.ds(j * _L, _L)]
                    s_hi = lax.bitcast_convert_type(s2 & himask, jnp.float32)
                    s_lo = lax.bitcast_convert_type(s2 << 16, jnp.float32)
                    o_hi = lax.bitcast_convert_type(o2 & himask, jnp.float32)
                    o_lo = lax.bitcast_convert_type(o2 << 16, jnp.float32)
                    acc = acc + s_lo * o_lo
                    acc = acc + s_hi * o_hi
                for sh in (8, 4, 2, 1):
                    sbuf[pl.ds(0, _L)] = acc
                    sbuf[pl.ds(_L, _L)] = acc
                    acc = acc + sbuf[pl.ds(sh, _L)]
                score_vec = jnp.where(lane == i2, acc, score_vec)
            scb[slot, pl.ds(g * _L, _L)] = score_vec
            return carry

        lax.fori_loop(0, _C // _L, group_body, 0)

    def super_body(s, carry):
        k0 = 8 * s
        for m in range(8):
            k = k0 + m
            p = m % 4
            bpar = 0 if m < 4 else 1
            slot = m % 4
            b = 2 * s + bpar

            if m == 2:
                @pl.when(s > 0)
                def _():
                    wait_idx(1)

            nslot = (m + 2) % 4
            npar = 0 if (m + 2 < 4 or m >= 6) else 1
            if m < 6:
                issue_streams(npar, nslot, (m + 2) % 4)
            elif m == 6:
                @pl.when(k + 2 < _NCH_W)
                def _():
                    wait_idx(0)
                    issue_streams(npar, nslot, (m + 2) % 4)
            else:
                @pl.when(k + 2 < _NCH_W)
                def _():
                    issue_streams(npar, nslot, (m + 2) % 4)

            if m == 2:
                @pl.when(2 * s + 2 < _NB)
                def _():
                    issue_idx(0, 8 * s + 8)
            if m == 6:
                @pl.when(2 * s + 3 < _NB)
                def _():
                    issue_idx(1, 8 * s + 12)

            wait_rows(p)
            compute_chunk(p, scb_bufs[bpar], slot)

            if slot == 3:
                @pl.when(b >= 2)
                def _():
                    pltpu.make_async_copy(scb_bufs[bpar],
                                          out_hbm.at[wid, pl.ds(0, 4)],
                                          out_sems[bpar]).wait()
                pltpu.async_copy(scb_bufs[bpar],
                                 out_hbm.at[wid, pl.ds(4 * b, 4)],
                                 out_sems[bpar])
        return carry

    lax.fori_loop(0, _NCH_W // 8, super_body, 0)

    pltpu.make_async_copy(scb_a, out_hbm.at[wid, pl.ds(0, 4)], osem_a).wait()
    pltpu.make_async_copy(scb_b, out_hbm.at[wid, pl.ds(0, 4)], osem_b).wait()


_sc_scores = functools.partial(
    pl.kernel,
    mesh=plsc.VectorSubcoreMesh(core_axis_name="c", subcore_axis_name="s"),
    out_type=jax.ShapeDtypeStruct((_NW, _NCH_W, _C), jnp.float32),
    scratch_types=[
        pltpu.VMEM((4, 2 * _C), jnp.int32),
        pltpu.VMEM((4, 2 * _C), jnp.int32),
        pltpu.VMEM((2 * _C, _HW), jnp.int32),
        pltpu.VMEM((2 * _C, _HW), jnp.int32),
        pltpu.VMEM((2 * _C, _HW), jnp.int32),
        pltpu.VMEM((2 * _C, _HW), jnp.int32),
        pltpu.VMEM((4, _C), jnp.float32),
        pltpu.VMEM((4, _C), jnp.float32),
        pltpu.VMEM((2 * _L,), jnp.float32),
        pltpu.SemaphoreType.DMA,
        pltpu.SemaphoreType.DMA,
        pltpu.SemaphoreType.DMA,
        pltpu.SemaphoreType.DMA,
        pltpu.SemaphoreType.DMA,
        pltpu.SemaphoreType.DMA,
        pltpu.SemaphoreType.DMA,
        pltpu.SemaphoreType.DMA,
    ],
)(_sc_body)


def _tab_body(e_ref, w_ref, o_ref):
    i = pl.program_id(0)
    x = e_ref[...]
    wrow = w_ref[pl.ds(jnp.maximum(i - 1, 0), 1), :]
    fac = jnp.where(i == 0, jnp.ones((1, _H), jnp.float32), wrow)
    x = x * fac
    lo = x[:, : _H // 2].astype(jnp.bfloat16)
    hi = x[:, _H // 2:].astype(jnp.bfloat16)
    lo_u = lax.bitcast_convert_type(lo, jnp.uint16).astype(jnp.uint32)
    hi_u = lax.bitcast_convert_type(hi, jnp.uint16).astype(jnp.uint32)
    o_ref[...] = lax.bitcast_convert_type(lo_u | (hi_u << 16), jnp.int32)


def _loss_body(s_ref, l_ref, e_ref, w_ref, out_ref):
    s = s_ref[...]
    lbl = l_ref[...]
    bce = jnp.maximum(s, 0.0) - s * lbl + jnp.log1p(jnp.exp(-jnp.abs(s)))
    predict = jnp.sum(bce) * (1.0 / _T)
    e = e_ref[...]
    w = w_ref[...]
    reg = (jnp.sum(e * e) * (1.0 / (_N * _H))
           + jnp.sum(w * w) * (1.0 / (_R * _H)))
    out_ref[...] = jnp.full((1, 1), predict + _REG * reg, jnp.float32)


def kernel(embed, w_relation, triplets, labels):
    heads = triplets[:, 0]
    rels = triplets[:, 1]
    tails = triplets[:, 2]
    z = jnp.zeros((_TP - _T,), jnp.int32)
    heads_p = jnp.concatenate([heads, z]).reshape(_NCH, _C)
    tails_p = jnp.concatenate([tails, z]).reshape(_NCH, _C)
    rels_p = jnp.concatenate([rels, z])
    ctab = pl.pallas_call(
        _tab_body,
        grid=(_R + 1,),
        in_specs=[pl.BlockSpec((_N, _H), lambda i: (0, 0)),
                  pl.BlockSpec((_R, _H), lambda i: (0, 0))],
        out_specs=pl.BlockSpec((_N, _HW), lambda i: (i, 0)),
        out_shape=jax.ShapeDtypeStruct(((_R + 1) * _N, _HW), jnp.int32),
    )(embed, w_relation)
    owidx_p = (_N + rels_p.reshape(_NCH, _C) * _N + tails_p)
    ht = jnp.concatenate([heads_p, owidx_p], axis=1)
    hidx_wm = ht.reshape(_NCH_W, _NW, 2 * _C).transpose(1, 0, 2)
    scores_wm = _sc_scores(hidx_wm, ctab)
    scores = scores_wm.transpose(1, 0, 2).reshape(-1)[:_T]
    out = pl.pallas_call(
        _loss_body,
        out_shape=jax.ShapeDtypeStruct((1, 1), jnp.float32),
    )(scores.reshape(_T // 128, 128), labels.reshape(_T // 128, 128),
      embed, w_relation)
    return out[0, 0]


# speedup vs baseline: 1.0045x; 1.0045x over previous
import functools

import jax
import jax.numpy as jnp
from jax import lax
from jax.experimental import pallas as pl
from jax.experimental.pallas import tpu as pltpu
from jax.experimental.pallas import tpu_sc as plsc

_T = 800000
_N = 10000
_H = 256
_HW = _H // 2
_R = 16
_REG = 0.01
_C = 64
_NW = 32
_L = 16
_NCH_W = 392
_NB = _NCH_W // 4
_NCH = _NCH_W * _NW
_TP = _NCH * _C


def _sc_body(hidx_hbm, ctab_hbm, out_hbm,
             h_a, h_b, rows_a, rows_b,
             scb_a, scb_b, sbuf,
             sem_a, sem_b, sem_ia, sem_ib, osem_a, osem_b):
    wid = lax.axis_index("s") * 2 + lax.axis_index("c")
    lane = lax.broadcasted_iota(jnp.int32, (_L,), 0)

    h_bufs = (h_a, h_b)
    row_bufs = (rows_a, rows_b)
    row_sems = (sem_a, sem_b)
    scb_bufs = (scb_a, scb_b)
    out_sems = (osem_a, osem_b)

    pltpu.sync_copy(hidx_hbm.at[wid, pl.ds(0, 4)], h_a)
    pltpu.sync_copy(hidx_hbm.at[wid, pl.ds(4, 4)], h_b)
    pltpu.async_copy(ctab_hbm.at[h_a.at[0]], rows_a, sem_a)

    def wait_idx(par):
        sem = sem_ia if par == 0 else sem_ib
        pltpu.make_async_copy(hidx_hbm.at[wid, pl.ds(0, 4)],
                              h_bufs[par], sem).wait()

    def issue_idx(par, batch_base):
        sem = sem_ia if par == 0 else sem_ib
        pltpu.async_copy(hidx_hbm.at[wid, pl.ds(batch_base, 4)],
                         h_bufs[par], sem)

    def issue_streams(src_par, slot, dst_p):
        pltpu.async_copy(ctab_hbm.at[h_bufs[src_par].at[slot]],
                         row_bufs[dst_p], row_sems[dst_p])

    def wait_rows(p):
        pltpu.make_async_copy(ctab_hbm.at[pl.ds(0, 2 * _C)],
                              row_bufs[p], row_sems[p]).wait()

    def compute_chunk(p, scb, slot):
        rows_c = row_bufs[p]

        def group_body(g, carry):
            score_vec = jnp.zeros((_L,), jnp.float32)
            himask = jnp.full((_L,), -65536, jnp.int32)
            for i2 in range(_L):
                t = g * _L + i2
                acc = jnp.zeros((_L,), jnp.float32)
                for j in range(_HW // _L):
                    s2 = rows_c[t, pl.ds(j * _L, _L)]
                    o2 = rows_c[_C + t, pl.ds(j * _L, _L)]
                    s_hi = lax.bitcast_convert_type(s2 & himask, jnp.float32)
                    s_lo = lax.bitcast_convert_type(s2 << 16, jnp.float32)
                    o_hi = lax.bitcast_convert_type(o2 & himask, jnp.float32)
                    o_lo = lax.bitcast_convert_type(o2 << 16, jnp.float32)
                    acc = acc + s_lo * o_lo
                    acc = acc + s_hi * o_hi
                for sh in (8, 4, 2, 1):
                    sbuf[pl.ds(0, _L)] = acc
                    sbuf[pl.ds(_L, _L)] = acc
                    acc = acc + sbuf[pl.ds(sh, _L)]
                score_vec = jnp.where(lane == i2, acc, score_vec)
            scb[slot, pl.ds(g * _L, _L)] = score_vec
            return carry

        lax.fori_loop(0, _C // _L, group_body, 0)

    def super_body(s, carry):
        k0 = 8 * s
        for m in range(8):
            k = k0 + m
            p = m % 2
            bpar = 0 if m < 4 else 1
            slot = m % 4
            b = 2 * s + bpar

            if m == 3:
                @pl.when(s > 0)
                def _():
                    wait_idx(1)

            nslot = (m + 1) % 4
            npar = 0 if (m + 1 < 4 or m == 7) else 1
            if m < 7:
                issue_streams(npar, nslot, 1 - p)
            else:
                @pl.when(k + 1 < _NCH_W)
                def _():
                    wait_idx(0)
                    issue_streams(npar, nslot, 1 - p)

            if m == 3:
                @pl.when(2 * s + 2 < _NB)
                def _():
                    issue_idx(0, 8 * s + 8)
            if m == 7:
                @pl.when(2 * s + 3 < _NB)
                def _():
                    issue_idx(1, 8 * s + 12)

            wait_rows(p)
            compute_chunk(p, scb_bufs[bpar], slot)

            if slot == 3:
                @pl.when(b >= 2)
                def _():
                    pltpu.make_async_copy(scb_bufs[bpar],
                                          out_hbm.at[wid, pl.ds(0, 4)],
                                          out_sems[bpar]).wait()
                pltpu.async_copy(scb_bufs[bpar],
                                 out_hbm.at[wid, pl.ds(4 * b, 4)],
                                 out_sems[bpar])
        return carry

    lax.fori_loop(0, _NCH_W // 8, super_body, 0)

    pltpu.make_async_copy(scb_a, out_hbm.at[wid, pl.ds(0, 4)], osem_a).wait()
    pltpu.make_async_copy(scb_b, out_hbm.at[wid, pl.ds(0, 4)], osem_b).wait()


_sc_scores = functools.partial(
    pl.kernel,
    mesh=plsc.VectorSubcoreMesh(core_axis_name="c", subcore_axis_name="s"),
    out_type=jax.ShapeDtypeStruct((_NW, _NCH_W, _C), jnp.float32),
    scratch_types=[
        pltpu.VMEM((4, 2 * _C), jnp.int32),
        pltpu.VMEM((4, 2 * _C), jnp.int32),
        pltpu.VMEM((2 * _C, _HW), jnp.int32),
        pltpu.VMEM((2 * _C, _HW), jnp.int32),
        pltpu.VMEM((4, _C), jnp.float32),
        pltpu.VMEM((4, _C), jnp.float32),
        pltpu.VMEM((2 * _L,), jnp.float32),
        pltpu.SemaphoreType.DMA,
        pltpu.SemaphoreType.DMA,
        pltpu.SemaphoreType.DMA,
        pltpu.SemaphoreType.DMA,
        pltpu.SemaphoreType.DMA,
        pltpu.SemaphoreType.DMA,
    ],
)(_sc_body)


def _tab_body(e_ref, w_ref, o_ref):
    i = pl.program_id(0)
    x = e_ref[...]
    wrow = w_ref[pl.ds(jnp.maximum(i - 1, 0), 1), :]
    fac = jnp.where(i == 0, jnp.ones((1, _H), jnp.float32), wrow)
    x = x * fac
    lo = x[:, : _H // 2].astype(jnp.bfloat16)
    hi = x[:, _H // 2:].astype(jnp.bfloat16)
    lo_u = lax.bitcast_convert_type(lo, jnp.uint16).astype(jnp.uint32)
    hi_u = lax.bitcast_convert_type(hi, jnp.uint16).astype(jnp.uint32)
    o_ref[...] = lax.bitcast_convert_type(lo_u | (hi_u << 16), jnp.int32)


def _loss_body(s_ref, l_ref, e_ref, w_ref, out_ref):
    s = s_ref[...]
    lbl = l_ref[...]
    bce = jnp.maximum(s, 0.0) - s * lbl + jnp.log1p(jnp.exp(-jnp.abs(s)))
    predict = jnp.sum(bce) * (1.0 / _T)
    e = e_ref[...]
    w = w_ref[...]
    reg = (jnp.sum(e * e) * (1.0 / (_N * _H))
           + jnp.sum(w * w) * (1.0 / (_R * _H)))
    out_ref[...] = jnp.full((1, 1), predict + _REG * reg, jnp.float32)


def kernel(embed, w_relation, triplets, labels):
    heads = triplets[:, 0]
    rels = triplets[:, 1]
    tails = triplets[:, 2]
    z = jnp.zeros((_TP - _T,), jnp.int32)
    heads_p = jnp.concatenate([heads, z]).reshape(_NCH, _C)
    tails_p = jnp.concatenate([tails, z]).reshape(_NCH, _C)
    rels_p = jnp.concatenate([rels, z])
    ctab = pl.pallas_call(
        _tab_body,
        grid=(_R + 1,),
        in_specs=[pl.BlockSpec((_N, _H), lambda i: (0, 0)),
                  pl.BlockSpec((_R, _H), lambda i: (0, 0))],
        out_specs=pl.BlockSpec((_N, _HW), lambda i: (i, 0)),
        out_shape=jax.ShapeDtypeStruct(((_R + 1) * _N, _HW), jnp.int32),
    )(embed, w_relation)
    owidx_p = (_N + rels_p.reshape(_NCH, _C) * _N + tails_p)
    ht = jnp.concatenate([heads_p, owidx_p], axis=1)
    hidx_wm = ht.reshape(_NCH_W, _NW, 2 * _C).transpose(1, 0, 2)
    scores_wm = _sc_scores(hidx_wm, ctab)
    scores = scores_wm.transpose(1, 0, 2).reshape(-1)[:_T]
    out = pl.pallas_call(
        _loss_body,
        out_shape=jax.ShapeDtypeStruct((1, 1), jnp.float32),
    )(scores.reshape(_T // 128, 128), labels.reshape(_T // 128, 128),
      embed, w_relation)
    return out[0, 0]
